# baseline (speedup 1.0000x reference)
import functools

import jax
import jax.numpy as jnp
from jax import lax
from jax.experimental import pallas as pl
from jax.experimental.pallas import tpu as pltpu
from jax.experimental.pallas import tpu_sc as plsc

B = 4096 * 200
SPLIT = B // 2
NW = 32
B_PER_W = SPLIT // NW
CHUNK = 128
N_CHUNKS = B_PER_W // CHUNK
N_ITERS = N_CHUNKS // 4

TC_BLK = 1024
N_TC_BLKS = (B - SPLIT) // TC_BLK


def _sc_body(tab_hbm, c_hbm, out_hbm, idx_v,
             buf0, buf1, buf2, buf3,
             g0, g1, g2, g3, w0, w1, w2, w3):
    bufs = (buf0, buf1, buf2, buf3)
    gsems = (g0, g1, g2, g3)
    wsems = (w0, w1, w2, w3)
    wid = lax.axis_index("s") * 2 + lax.axis_index("c")
    base = wid * B_PER_W
    pltpu.sync_copy(c_hbm.at[wid], idx_v)

    def gather(k, s):
        return pltpu.make_async_copy(tab_hbm.at[idx_v.at[k]], bufs[s], gsems[s])

    def write(k, s):
        return pltpu.make_async_copy(
            bufs[s], out_hbm.at[pl.ds(base + k * CHUNK, CHUNK)], wsems[s])

    gather(0, 0).start()
    gather(1, 1).start()

    def body4(j, carry):
        for b in range(4):
            k = 4 * j + b
            s = b
            sp = (b + 2) % 4

            if b >= 2:
                write(k - 2, sp).wait()
            else:
                @pl.when(j > 0)
                def _():
                    write(k - 2, sp).wait()

            if b < 2:
                gather(k + 2, sp).start()
            else:
                @pl.when(j < N_ITERS - 1)
                def _():
                    gather(k + 2, sp).start()

            gather(k, s).wait()
            write(k, s).start()
        return carry

    lax.fori_loop(0, N_ITERS, body4, 0)
    write(N_CHUNKS - 2, 2).wait()
    write(N_CHUNKS - 1, 3).wait()


def _tc_body(c_ref, tab_ref, outin_ref, out_ref):
    cb = c_ref[...]
    iot = lax.broadcasted_iota(jnp.int32, (TC_BLK, 256), 1)
    oh = (cb == iot).astype(jnp.float32)
    out_ref[...] = jnp.dot(oh, tab_ref[...],
                           preferred_element_type=jnp.float32)


@jax.jit
def kernel(x, street_emb, action_emb, position_emb):
    x32 = x.reshape(B, 7).astype(jnp.int32)
    c = (x32[:, 1] + 4 * x32[:, 0] + 16 * x32[:, 6] + 64 * x32[:, 5])

    i = jnp.arange(256, dtype=jnp.int32)
    tab = jnp.concatenate(
        (
            street_emb[i & 3],
            street_emb[(i >> 2) & 3],
            action_emb[(i >> 4) & 3],
            position_emb[(i >> 6) & 3],
        ),
        axis=1,
    )

    c_sc = c[:SPLIT].reshape(NW, N_CHUNKS, CHUNK)
    c_tc = c[SPLIT:].reshape(B - SPLIT, 1)

    mesh = plsc.VectorSubcoreMesh(core_axis_name="c", subcore_axis_name="s")
    sc_run = functools.partial(
        pl.kernel,
        mesh=mesh,
        out_type=jax.ShapeDtypeStruct((B, 128), jnp.float32),
        scratch_types=[
            pltpu.VMEM((N_CHUNKS, CHUNK), jnp.int32),
            pltpu.VMEM((CHUNK, 128), jnp.float32),
            pltpu.VMEM((CHUNK, 128), jnp.float32),
            pltpu.VMEM((CHUNK, 128), jnp.float32),
            pltpu.VMEM((CHUNK, 128), jnp.float32),
            pltpu.SemaphoreType.DMA,
            pltpu.SemaphoreType.DMA,
            pltpu.SemaphoreType.DMA,
            pltpu.SemaphoreType.DMA,
            pltpu.SemaphoreType.DMA,
            pltpu.SemaphoreType.DMA,
            pltpu.SemaphoreType.DMA,
            pltpu.SemaphoreType.DMA,
        ],
    )(_sc_body)
    out_sc = sc_run(tab, c_sc)

    n_sc_blks = SPLIT // TC_BLK
    out = pl.pallas_call(
        _tc_body,
        grid=(N_TC_BLKS,),
        in_specs=[
            pl.BlockSpec((TC_BLK, 1), lambda i: (i, 0)),
            pl.BlockSpec((256, 128), lambda i: (0, 0)),
            pl.BlockSpec((TC_BLK, 128), lambda i: (n_sc_blks + i, 0)),
        ],
        out_specs=pl.BlockSpec((TC_BLK, 128), lambda i: (n_sc_blks + i, 0)),
        out_shape=jax.ShapeDtypeStruct((B, 128), jnp.float32),
        input_output_aliases={2: 0},
    )(c_tc, tab, out_sc)
    return out.reshape(4096, 200, 128)


# speedup vs baseline: 1.0649x; 1.0649x over previous
import functools

import jax
import jax.numpy as jnp
from jax import lax
from jax.experimental import pallas as pl
from jax.experimental.pallas import tpu as pltpu
from jax.experimental.pallas import tpu_sc as plsc

B = 4096 * 200
SPLIT = B // 2
NW = 32
B_PER_W = SPLIT // 16
CHUNK = 128
N_CHUNKS = B_PER_W // CHUNK
N_ITERS = N_CHUNKS // 4

TC_BLK = 1024
N_TC_BLKS = (B - SPLIT) // TC_BLK


def _sc_body(tab_hbm, c_hbm, out_hbm, idx_v,
             buf0, buf1, buf2, buf3,
             g0, g1, g2, g3, w0, w1, w2, w3):
    bufs = (buf0, buf1, buf2, buf3)
    gsems = (g0, g1, g2, g3)
    wsems = (w0, w1, w2, w3)
    wid = lax.axis_index("s")
    base = wid * B_PER_W
    pltpu.sync_copy(c_hbm.at[wid], idx_v)

    def gather(k, s):
        return pltpu.make_async_copy(tab_hbm.at[idx_v.at[k]], bufs[s], gsems[s])

    def write(k, s):
        return pltpu.make_async_copy(
            bufs[s], out_hbm.at[pl.ds(base + k * CHUNK, CHUNK)], wsems[s])

    gather(0, 0).start()
    gather(1, 1).start()

    def body4(j, carry):
        for b in range(4):
            k = 4 * j + b
            s = b
            sp = (b + 2) % 4

            if b >= 2:
                write(k - 2, sp).wait()
            else:
                @pl.when(j > 0)
                def _():
                    write(k - 2, sp).wait()

            if b < 2:
                gather(k + 2, sp).start()
            else:
                @pl.when(j < N_ITERS - 1)
                def _():
                    gather(k + 2, sp).start()

            gather(k, s).wait()
            write(k, s).start()
        return carry

    lax.fori_loop(0, N_ITERS, body4, 0)
    write(N_CHUNKS - 2, 2).wait()
    write(N_CHUNKS - 1, 3).wait()


def _tc_body(c_ref, tab_ref, outin_ref, out_ref):
    cb = c_ref[...]
    iot = lax.broadcasted_iota(jnp.int32, (TC_BLK, 256), 1)
    oh = (cb == iot).astype(jnp.float32)
    out_ref[...] = jnp.dot(oh, tab_ref[...],
                           preferred_element_type=jnp.float32)


@jax.jit
def kernel(x, street_emb, action_emb, position_emb):
    x32 = x.reshape(B, 7).astype(jnp.int32)
    c = (x32[:, 1] + 4 * x32[:, 0] + 16 * x32[:, 6] + 64 * x32[:, 5])

    i = jnp.arange(256, dtype=jnp.int32)
    tab = jnp.concatenate(
        (
            street_emb[i & 3],
            street_emb[(i >> 2) & 3],
            action_emb[(i >> 4) & 3],
            position_emb[(i >> 6) & 3],
        ),
        axis=1,
    )

    c_sc = c[:SPLIT].reshape(16, N_CHUNKS, CHUNK)
    c_tc = c[SPLIT:].reshape(B - SPLIT, 1)

    mesh = plsc.VectorSubcoreMesh(core_axis_name="c", subcore_axis_name="s", num_cores=1)
    sc_run = functools.partial(
        pl.kernel,
        mesh=mesh,
        out_type=jax.ShapeDtypeStruct((B, 128), jnp.float32),
        scratch_types=[
            pltpu.VMEM((N_CHUNKS, CHUNK), jnp.int32),
            pltpu.VMEM((CHUNK, 128), jnp.float32),
            pltpu.VMEM((CHUNK, 128), jnp.float32),
            pltpu.VMEM((CHUNK, 128), jnp.float32),
            pltpu.VMEM((CHUNK, 128), jnp.float32),
            pltpu.SemaphoreType.DMA,
            pltpu.SemaphoreType.DMA,
            pltpu.SemaphoreType.DMA,
            pltpu.SemaphoreType.DMA,
            pltpu.SemaphoreType.DMA,
            pltpu.SemaphoreType.DMA,
            pltpu.SemaphoreType.DMA,
            pltpu.SemaphoreType.DMA,
        ],
    )(_sc_body)
    out_sc = sc_run(tab, c_sc)

    n_sc_blks = SPLIT // TC_BLK
    out = pl.pallas_call(
        _tc_body,
        grid=(N_TC_BLKS,),
        in_specs=[
            pl.BlockSpec((TC_BLK, 1), lambda i: (i, 0)),
            pl.BlockSpec((256, 128), lambda i: (0, 0)),
            pl.BlockSpec(memory_space=pl.ANY),
        ],
        out_specs=pl.BlockSpec((TC_BLK, 128), lambda i: (n_sc_blks + i, 0)),
        out_shape=jax.ShapeDtypeStruct((B, 128), jnp.float32),
        input_output_aliases={2: 0},
    )(c_tc, tab, out_sc)
    return out.reshape(4096, 200, 128)
